# Initial kernel scaffold; baseline (speedup 1.0000x reference)
import jax
import jax.numpy as jnp
from jax.experimental import pallas as pl


def kernel(hidden_states, W_router, router_scale, W_gate, W_up, W_down):
    raise NotImplementedError("write your pallas kernel here")


# speedup vs baseline: 3.0749x; 3.0749x over previous
import functools

import jax
import jax.numpy as jnp
from jax.experimental import pallas as pl
from jax.experimental.pallas import tpu as pltpu

NUM_EXPERTS = 16
TOP_K = 2
D_MODEL = 1024
D_FF = 512
NUM_TOKENS = 1024
EPG = NUM_EXPERTS // TOP_K


def _moe_body(x_ref, wr_ref, rs_ref, wg_ref, wu_ref, wd_ref, out_ref, wm_ref):
    e = pl.program_id(0)

    @pl.when(e == 0)
    def _router():
        x = x_ref[...]
        gating = jax.lax.dot_general(
            x, wr_ref[...], (((1,), (1,)), ((), ())),
            preferred_element_type=jnp.float32)
        m = jnp.max(gating, axis=1, keepdims=True)
        ex = jnp.exp(gating - m)
        scores = ex / jnp.sum(ex, axis=1, keepdims=True)
        lane = jax.lax.broadcasted_iota(jnp.int32, (NUM_TOKENS, NUM_EXPERTS), 1)
        rs = rs_ref[...]
        wm = jnp.zeros((NUM_TOKENS, NUM_EXPERTS), jnp.float32)
        for g in range(TOP_K):
            in_grp = (lane >= g * EPG) & (lane < (g + 1) * EPG)
            sg = jnp.where(in_grp, scores, -1.0)
            mx = jnp.max(sg, axis=1, keepdims=True)
            idx = jnp.min(jnp.where((sg == mx) & in_grp, lane, NUM_EXPERTS),
                          axis=1, keepdims=True)
            sel = lane == idx
            rsel = jnp.sum(jnp.where(sel, rs, 0.0), axis=1, keepdims=True)
            wm = wm + jnp.where(sel, mx * rsel, 0.0)
        wm_ref[...] = wm

    x = x_ref[...]
    wg = wg_ref[0]
    wu = wu_ref[0]
    wd = wd_ref[0]
    g = jax.lax.dot_general(x, wg, (((1,), (0,)), ((), ())),
                            preferred_element_type=jnp.float32)
    u = jax.lax.dot_general(x, wu, (((1,), (0,)), ((), ())),
                            preferred_element_type=jnp.float32)
    h = g * jax.lax.logistic(g) * u
    y = jax.lax.dot_general(h, wd, (((1,), (0,)), ((), ())),
                            preferred_element_type=jnp.float32)
    lane = jax.lax.broadcasted_iota(jnp.int32, (NUM_TOKENS, NUM_EXPERTS), 1)
    w_col = jnp.sum(jnp.where(lane == e, wm_ref[...], 0.0), axis=1,
                    keepdims=True)

    @pl.when(e == 0)
    def _init():
        out_ref[...] = jnp.zeros_like(out_ref)

    out_ref[...] += w_col * y


@jax.jit
def kernel(hidden_states, W_router, router_scale, W_gate, W_up, W_down):
    return pl.pallas_call(
        _moe_body,
        grid=(NUM_EXPERTS,),
        in_specs=[
            pl.BlockSpec((NUM_TOKENS, D_MODEL), lambda e: (0, 0)),
            pl.BlockSpec((NUM_EXPERTS, D_MODEL), lambda e: (0, 0)),
            pl.BlockSpec((1, NUM_EXPERTS), lambda e: (0, 0)),
            pl.BlockSpec((1, D_MODEL, D_FF), lambda e: (e, 0, 0)),
            pl.BlockSpec((1, D_MODEL, D_FF), lambda e: (e, 0, 0)),
            pl.BlockSpec((1, D_FF, D_MODEL), lambda e: (e, 0, 0)),
        ],
        out_specs=pl.BlockSpec((NUM_TOKENS, D_MODEL), lambda e: (0, 0)),
        out_shape=jax.ShapeDtypeStruct((NUM_TOKENS, D_MODEL), jnp.float32),
        scratch_shapes=[pltpu.VMEM((NUM_TOKENS, NUM_EXPERTS), jnp.float32)],
    )(hidden_states, W_router, router_scale.reshape(1, NUM_EXPERTS),
      W_gate, W_up, W_down)
